# baseline (speedup 1.0000x reference)
import functools

import jax
import jax.numpy as jnp
from jax import lax
from jax.experimental import pallas as pl
from jax.experimental.pallas import tpu as pltpu
from jax.experimental.pallas import tpu_sc as plsc

_TEMP = 0.05
_B = 1024
_D = 128
_WLANES = 128


def _sc_gather_body(feat_hbm, w2_hbm, tgt_hbm, ind_hbm, rows_out, w_out,
                    tidx_v, iidx_v, iwrow_v, rows_v, wrows_v, sem1, sem2,
                    *, n_workers, b_per_w):
    wid = lax.axis_index("s") * 2 + lax.axis_index("c")
    base = wid * b_per_w
    pltpu.sync_copy(tgt_hbm.at[pl.ds(base, b_per_w)], tidx_v)
    pltpu.sync_copy(ind_hbm.at[pl.ds(base, b_per_w)], iidx_v)
    cp1 = pltpu.async_copy(feat_hbm.at[tidx_v], rows_v, sem1)
    for c in range(b_per_w // 16):
        v = iidx_v[pl.ds(c * 16, 16)]
        iwrow_v[pl.ds(c * 16, 16)] = lax.shift_right_logical(v, 7)
    cp2 = pltpu.async_copy(w2_hbm.at[iwrow_v], wrows_v, sem2)
    cp1.wait()
    cp2.wait()
    pltpu.sync_copy(rows_v, rows_out.at[pl.ds(base, b_per_w)])
    pltpu.sync_copy(wrows_v, w_out.at[pl.ds(base, b_per_w)])


def _sc_gather(features, w2, targets, inds):
    info = plsc.get_sparse_core_info()
    n_workers = info.num_cores * info.num_subcores
    b_per_w = _B // n_workers
    mesh = plsc.VectorSubcoreMesh(core_axis_name="c", subcore_axis_name="s")
    body = functools.partial(_sc_gather_body, n_workers=n_workers,
                             b_per_w=b_per_w)
    return pl.kernel(
        body, mesh=mesh,
        out_type=[jax.ShapeDtypeStruct((_B, _D), jnp.float32),
                  jax.ShapeDtypeStruct((_B, _WLANES), jnp.float32)],
        scratch_types=[
            pltpu.VMEM((b_per_w,), jnp.int32),
            pltpu.VMEM((b_per_w,), jnp.int32),
            pltpu.VMEM((b_per_w,), jnp.int32),
            pltpu.VMEM((b_per_w, _D), jnp.float32),
            pltpu.VMEM((b_per_w, _WLANES), jnp.float32),
            pltpu.SemaphoreType.DMA,
            pltpu.SemaphoreType.DMA,
        ],
    )(features, w2, targets, inds)


def _lse_kernel(x_ref, feat_ref, s_out_ref, xn_ref, s_ref, *, n_blocks):
    j = pl.program_id(0)

    @pl.when(j == 0)
    def _init():
        x = x_ref[...]
        nrm = jnp.sqrt(jnp.sum(x * x, axis=1, keepdims=True))
        scale = 1.4426950408889634 / _TEMP
        xn_ref[...] = (x * (scale / jnp.maximum(nrm, 1e-12))
                       ).astype(jnp.bfloat16)
        s_ref[...] = jnp.zeros_like(s_ref)

    logits2 = jax.lax.dot_general(
        xn_ref[...], feat_ref[...].astype(jnp.bfloat16),
        (((1,), (1,)), ((), ())),
        preferred_element_type=jnp.float32)
    s_ref[...] += jnp.sum(jnp.exp2(logits2), axis=1, keepdims=True)

    @pl.when(j == n_blocks - 1)
    def _fin():
        s_out_ref[...] = s_ref[...]


def _combine_kernel(x_ref, rows_ref, wrows_ref, ind_ref, s_ref, out_ref):
    x = x_ref[...]
    nrm = jnp.sqrt(jnp.sum(x * x, axis=1, keepdims=True))
    xn = x / jnp.maximum(nrm, 1e-12)
    tl = jnp.sum(xn * rows_ref[...], axis=1, keepdims=True) * (1.0 / _TEMP)
    lane = jax.lax.broadcasted_iota(jnp.int32, (_B, _WLANES), 1)
    w = jnp.sum(jnp.where(lane == ind_ref[...] % _WLANES, wrows_ref[...], 0.0),
                axis=1, keepdims=True)
    per = (jnp.log(s_ref[...]) - tl) * w
    out_ref[...] = jnp.full_like(out_ref, jnp.sum(per) / _B)


def _tc_part(inputs, features, tgt_rows, wrows, inds):
    k_total = features.shape[0]
    blk_k = 2000
    n_blocks = k_total // blk_k

    s = pl.pallas_call(
        functools.partial(_lse_kernel, n_blocks=n_blocks),
        grid=(n_blocks,),
        in_specs=[
            pl.BlockSpec((_B, _D), lambda j: (0, 0)),
            pl.BlockSpec((blk_k, _D), lambda j: (j, 0)),
        ],
        out_specs=pl.BlockSpec((_B, 1), lambda j: (0, 0)),
        out_shape=jax.ShapeDtypeStruct((_B, 1), jnp.float32),
        scratch_shapes=[
            pltpu.VMEM((_B, _D), jnp.bfloat16),
            pltpu.VMEM((_B, 1), jnp.float32),
        ],
        compiler_params=pltpu.CompilerParams(
            dimension_semantics=("arbitrary",)),
    )(inputs, features)

    out = pl.pallas_call(
        _combine_kernel,
        in_specs=[
            pl.BlockSpec((_B, _D), lambda: (0, 0)),
            pl.BlockSpec((_B, _D), lambda: (0, 0)),
            pl.BlockSpec((_B, _WLANES), lambda: (0, 0)),
            pl.BlockSpec((_B, 1), lambda: (0, 0)),
            pl.BlockSpec((_B, 1), lambda: (0, 0)),
        ],
        out_specs=pl.BlockSpec((8, 128), lambda: (0, 0)),
        out_shape=jax.ShapeDtypeStruct((8, 128), jnp.float32),
    )(inputs, tgt_rows, wrows, inds.astype(jnp.int32).reshape(_B, 1), s)
    return out[0, 0]


def kernel(inputs, targets, inds, features, weight):
    k_total = features.shape[0]
    n_wrows = pl.cdiv(k_total, _WLANES)
    wpad = jnp.pad(weight, (0, n_wrows * _WLANES - k_total))
    tgt_rows, wrows = _sc_gather(features,
                                 wpad.reshape(n_wrows, _WLANES),
                                 targets.astype(jnp.int32),
                                 inds.astype(jnp.int32))
    return _tc_part(inputs, features, tgt_rows, wrows, inds)


# speedup vs baseline: 1.1113x; 1.1113x over previous
import functools

import jax
import jax.numpy as jnp
from jax import lax
from jax.experimental import pallas as pl
from jax.experimental.pallas import tpu as pltpu
from jax.experimental.pallas import tpu_sc as plsc

_TEMP = 0.05
_B = 1024
_D = 128
_WLANES = 128


def _sc_gather_body(feat_hbm, w2_hbm, tgt_hbm, ind_hbm, rows_out, w_out,
                    tidx_v, iidx_v, iwrow_v, rows_v, wrows_v, sem1, sem2,
                    *, n_workers, b_per_w):
    wid = lax.axis_index("s") * 2 + lax.axis_index("c")
    base = wid * b_per_w
    pltpu.sync_copy(tgt_hbm.at[pl.ds(base, b_per_w)], tidx_v)
    pltpu.sync_copy(ind_hbm.at[pl.ds(base, b_per_w)], iidx_v)
    cp1 = pltpu.async_copy(feat_hbm.at[tidx_v], rows_v, sem1)
    for c in range(b_per_w // 16):
        v = iidx_v[pl.ds(c * 16, 16)]
        iwrow_v[pl.ds(c * 16, 16)] = lax.shift_right_logical(v, 7)
    cp2 = pltpu.async_copy(w2_hbm.at[iwrow_v], wrows_v, sem2)
    cp1.wait()
    cp2.wait()
    pltpu.sync_copy(rows_v, rows_out.at[pl.ds(base, b_per_w)])
    pltpu.sync_copy(wrows_v, w_out.at[pl.ds(base, b_per_w)])


def _sc_gather(features, w2, targets, inds):
    info = plsc.get_sparse_core_info()
    n_workers = info.num_cores * info.num_subcores
    b_per_w = _B // n_workers
    mesh = plsc.VectorSubcoreMesh(core_axis_name="c", subcore_axis_name="s")
    body = functools.partial(_sc_gather_body, n_workers=n_workers,
                             b_per_w=b_per_w)
    return pl.kernel(
        body, mesh=mesh,
        out_type=[jax.ShapeDtypeStruct((_B, _D), jnp.float32),
                  jax.ShapeDtypeStruct((_B, _WLANES), jnp.float32)],
        scratch_types=[
            pltpu.VMEM((b_per_w,), jnp.int32),
            pltpu.VMEM((b_per_w,), jnp.int32),
            pltpu.VMEM((b_per_w,), jnp.int32),
            pltpu.VMEM((b_per_w, _D), jnp.float32),
            pltpu.VMEM((b_per_w, _WLANES), jnp.float32),
            pltpu.SemaphoreType.DMA,
            pltpu.SemaphoreType.DMA,
        ],
    )(features, w2, targets, inds)


def _lse_kernel(x_ref, feat_ref, s_out_ref, xn_ref, s_ref, *, n_blocks):
    j = pl.program_id(0)

    @pl.when(j == 0)
    def _init():
        x = x_ref[...]
        nrm = jnp.sqrt(jnp.sum(x * x, axis=1, keepdims=True))
        scale = 1.4426950408889634 / _TEMP
        xn_ref[...] = (x * (scale / jnp.maximum(nrm, 1e-12))
                       ).astype(jnp.bfloat16)
        s_ref[...] = jnp.zeros_like(s_ref)

    logits2 = jax.lax.dot_general(
        xn_ref[...], feat_ref[...].astype(jnp.bfloat16),
        (((1,), (1,)), ((), ())),
        preferred_element_type=jnp.float32)
    s_ref[...] += jnp.sum(jnp.exp2(logits2), axis=1, keepdims=True)

    @pl.when(j == n_blocks - 1)
    def _fin():
        s_out_ref[...] = s_ref[...]


def _combine_kernel(x_ref, rows_ref, wrows_ref, ind_ref, s_ref, out_ref):
    x = x_ref[...]
    nrm = jnp.sqrt(jnp.sum(x * x, axis=1, keepdims=True))
    xn = x / jnp.maximum(nrm, 1e-12)
    tl = jnp.sum(xn * rows_ref[...], axis=1, keepdims=True) * (1.0 / _TEMP)
    lane = jax.lax.broadcasted_iota(jnp.int32, (_B, _WLANES), 1)
    w = jnp.sum(jnp.where(lane == ind_ref[...] % _WLANES, wrows_ref[...], 0.0),
                axis=1, keepdims=True)
    per = (jnp.log(s_ref[...]) - tl) * w
    out_ref[...] = jnp.full_like(out_ref, jnp.sum(per) / _B)


def _tc_part(inputs, features, tgt_rows, wrows, inds):
    k_total = features.shape[0]
    blk_k = 4000
    n_blocks = k_total // blk_k

    s = pl.pallas_call(
        functools.partial(_lse_kernel, n_blocks=n_blocks),
        grid=(n_blocks,),
        in_specs=[
            pl.BlockSpec((_B, _D), lambda j: (0, 0)),
            pl.BlockSpec((blk_k, _D), lambda j: (j, 0)),
        ],
        out_specs=pl.BlockSpec((_B, 1), lambda j: (0, 0)),
        out_shape=jax.ShapeDtypeStruct((_B, 1), jnp.float32),
        scratch_shapes=[
            pltpu.VMEM((_B, _D), jnp.bfloat16),
            pltpu.VMEM((_B, 1), jnp.float32),
        ],
        compiler_params=pltpu.CompilerParams(
            dimension_semantics=("arbitrary",)),
    )(inputs, features)

    out = pl.pallas_call(
        _combine_kernel,
        in_specs=[
            pl.BlockSpec((_B, _D), lambda: (0, 0)),
            pl.BlockSpec((_B, _D), lambda: (0, 0)),
            pl.BlockSpec((_B, _WLANES), lambda: (0, 0)),
            pl.BlockSpec((_B, 1), lambda: (0, 0)),
            pl.BlockSpec((_B, 1), lambda: (0, 0)),
        ],
        out_specs=pl.BlockSpec((8, 128), lambda: (0, 0)),
        out_shape=jax.ShapeDtypeStruct((8, 128), jnp.float32),
    )(inputs, tgt_rows, wrows, inds.astype(jnp.int32).reshape(_B, 1), s)
    return out[0, 0]


def kernel(inputs, targets, inds, features, weight):
    k_total = features.shape[0]
    n_wrows = pl.cdiv(k_total, _WLANES)
    wpad = jnp.pad(weight, (0, n_wrows * _WLANES - k_total))
    tgt_rows, wrows = _sc_gather(features,
                                 wpad.reshape(n_wrows, _WLANES),
                                 targets.astype(jnp.int32),
                                 inds.astype(jnp.int32))
    return _tc_part(inputs, features, tgt_rows, wrows, inds)


# speedup vs baseline: 1.2144x; 1.0928x over previous
import functools

import jax
import jax.numpy as jnp
from jax import lax
from jax.experimental import pallas as pl
from jax.experimental.pallas import tpu as pltpu
from jax.experimental.pallas import tpu_sc as plsc

_TEMP = 0.05
_B = 1024
_D = 128
_WLANES = 128


def _sc_gather_body(feat_hbm, w2_hbm, tgt_hbm, ind_hbm, rows_out, w_out,
                    tidx_v, iidx_v, iwrow_v, rows_v, wrows_v, sem1, sem2,
                    *, n_workers, b_per_w):
    wid = lax.axis_index("s") * 2 + lax.axis_index("c")
    base = wid * b_per_w
    pltpu.sync_copy(tgt_hbm.at[pl.ds(base, b_per_w)], tidx_v)
    pltpu.sync_copy(ind_hbm.at[pl.ds(base, b_per_w)], iidx_v)
    cp1 = pltpu.async_copy(feat_hbm.at[tidx_v], rows_v, sem1)
    for c in range(b_per_w // 16):
        v = iidx_v[pl.ds(c * 16, 16)]
        iwrow_v[pl.ds(c * 16, 16)] = lax.shift_right_logical(v, 7)
    cp2 = pltpu.async_copy(w2_hbm.at[iwrow_v], wrows_v, sem2)
    cp1.wait()
    cp2.wait()
    pltpu.sync_copy(rows_v, rows_out.at[pl.ds(base, b_per_w)])
    pltpu.sync_copy(wrows_v, w_out.at[pl.ds(base, b_per_w)])


def _sc_gather(features, w2, targets, inds):
    info = plsc.get_sparse_core_info()
    n_workers = info.num_cores * info.num_subcores
    b_per_w = _B // n_workers
    mesh = plsc.VectorSubcoreMesh(core_axis_name="c", subcore_axis_name="s")
    body = functools.partial(_sc_gather_body, n_workers=n_workers,
                             b_per_w=b_per_w)
    return pl.kernel(
        body, mesh=mesh,
        out_type=[jax.ShapeDtypeStruct((_B, _D), jnp.float32),
                  jax.ShapeDtypeStruct((_B, _WLANES), jnp.float32)],
        scratch_types=[
            pltpu.VMEM((b_per_w,), jnp.int32),
            pltpu.VMEM((b_per_w,), jnp.int32),
            pltpu.VMEM((b_per_w,), jnp.int32),
            pltpu.VMEM((b_per_w, _D), jnp.float32),
            pltpu.VMEM((b_per_w, _WLANES), jnp.float32),
            pltpu.SemaphoreType.DMA,
            pltpu.SemaphoreType.DMA,
        ],
    )(features, w2, targets, inds)


def _lse_kernel(x_ref, feat_ref, s_out_ref, xn_ref, s_ref, *, n_blocks):
    j = pl.program_id(0)

    @pl.when(j == 0)
    def _init():
        x = x_ref[...]
        nrm = jnp.sqrt(jnp.sum(x * x, axis=1, keepdims=True))
        scale = 1.4426950408889634 / _TEMP
        xn_ref[...] = (x * (scale / jnp.maximum(nrm, 1e-12))
                       ).astype(jnp.bfloat16)
        s_ref[...] = jnp.zeros_like(s_ref)

    logits2 = jax.lax.dot_general(
        xn_ref[...], feat_ref[...].astype(jnp.bfloat16),
        (((1,), (1,)), ((), ())),
        preferred_element_type=jnp.float32)
    s_ref[...] += jnp.sum(jnp.exp2(logits2), axis=1, keepdims=True)

    @pl.when(j == n_blocks - 1)
    def _fin():
        s_out_ref[...] = s_ref[...]


def _combine_kernel(x_ref, rows_ref, wrows_ref, ind_ref, s_ref, out_ref):
    x = x_ref[...]
    nrm = jnp.sqrt(jnp.sum(x * x, axis=1, keepdims=True))
    xn = x / jnp.maximum(nrm, 1e-12)
    tl = jnp.sum(xn * rows_ref[...], axis=1, keepdims=True) * (1.0 / _TEMP)
    lane = jax.lax.broadcasted_iota(jnp.int32, (_B, _WLANES), 1)
    w = jnp.sum(jnp.where(lane == ind_ref[...] % _WLANES, wrows_ref[...], 0.0),
                axis=1, keepdims=True)
    per = (jnp.log(s_ref[...]) - tl) * w
    out_ref[...] = jnp.full_like(out_ref, jnp.sum(per) / _B)


def _tc_part(inputs, features, tgt_rows, wrows, inds):
    k_total = features.shape[0]
    blk_k = 8000
    n_blocks = k_total // blk_k

    s = pl.pallas_call(
        functools.partial(_lse_kernel, n_blocks=n_blocks),
        grid=(n_blocks,),
        in_specs=[
            pl.BlockSpec((_B, _D), lambda j: (0, 0)),
            pl.BlockSpec((blk_k, _D), lambda j: (j, 0)),
        ],
        out_specs=pl.BlockSpec((_B, 1), lambda j: (0, 0)),
        out_shape=jax.ShapeDtypeStruct((_B, 1), jnp.float32),
        scratch_shapes=[
            pltpu.VMEM((_B, _D), jnp.bfloat16),
            pltpu.VMEM((_B, 1), jnp.float32),
        ],
        compiler_params=pltpu.CompilerParams(
            dimension_semantics=("arbitrary",)),
    )(inputs, features)

    out = pl.pallas_call(
        _combine_kernel,
        in_specs=[
            pl.BlockSpec((_B, _D), lambda: (0, 0)),
            pl.BlockSpec((_B, _D), lambda: (0, 0)),
            pl.BlockSpec((_B, _WLANES), lambda: (0, 0)),
            pl.BlockSpec((_B, 1), lambda: (0, 0)),
            pl.BlockSpec((_B, 1), lambda: (0, 0)),
        ],
        out_specs=pl.BlockSpec((8, 128), lambda: (0, 0)),
        out_shape=jax.ShapeDtypeStruct((8, 128), jnp.float32),
    )(inputs, tgt_rows, wrows, inds.astype(jnp.int32).reshape(_B, 1), s)
    return out[0, 0]


def kernel(inputs, targets, inds, features, weight):
    k_total = features.shape[0]
    n_wrows = pl.cdiv(k_total, _WLANES)
    wpad = jnp.pad(weight, (0, n_wrows * _WLANES - k_total))
    tgt_rows, wrows = _sc_gather(features,
                                 wpad.reshape(n_wrows, _WLANES),
                                 targets.astype(jnp.int32),
                                 inds.astype(jnp.int32))
    return _tc_part(inputs, features, tgt_rows, wrows, inds)
